# baseline (speedup 1.0000x reference)
import functools

import jax
import jax.numpy as jnp
from jax import lax
from jax.experimental import pallas as pl
from jax.experimental.pallas import tpu as pltpu
from jax.experimental.pallas import tpu_sc as plsc

D = 128


def _mlp_host(layers, x):
    for W, b in layers[:-1]:
        x = jax.nn.silu(x @ W + b)
    W, b = layers[-1]
    return x @ W + b


def _adaln_host(p, x):
    gb = _mlp_host(p['gb'], x)
    gamma, beta = jnp.split(gb, 2, axis=-1)
    alpha = _mlp_host(p['alpha'], x)
    return gamma, beta, alpha


def _fold_messenger(gamma, beta, mlp):
    (W1, b1), (W2, b2), (W3, b3) = mlp
    W1s = gamma[:, None] * W1
    u1 = jnp.sum(W1s, axis=0)
    c1 = beta @ W1 + b1
    return W1s, u1, c1, W2, b2, W3, b3


_CH = 128


def _sc_gather(table, idx2d):
    n_slabs = idx2d.shape[0]
    info = plsc.get_sparse_core_info()
    nw = info.num_cores * info.num_subcores
    n_ch = n_slabs // nw
    B = n_slabs * _CH
    mesh = plsc.VectorSubcoreMesh(core_axis_name="c", subcore_axis_name="s")

    @functools.partial(
        pl.kernel,
        out_type=jax.ShapeDtypeStruct((B, D), jnp.float32),
        mesh=mesh,
        scratch_types=[
            pltpu.VMEM((_CH,), jnp.int32),
            pltpu.VMEM((_CH, D), jnp.float32),
        ],
    )
    def gather_k(table_hbm, idx_hbm, out_hbm, idx_v, rows_v):
        wid = lax.axis_index("s") * info.num_cores + lax.axis_index("c")

        def body(c, carry):
            slab = wid * n_ch + c
            pltpu.sync_copy(idx_hbm.at[slab], idx_v)
            pltpu.sync_copy(table_hbm.at[idx_v], rows_v)
            pltpu.sync_copy(rows_v, out_hbm.at[pl.ds(slab * _CH, _CH)])
            return carry

        lax.fori_loop(0, n_ch, body, 0)

    return gather_k(table, idx2d)


def _inv_std(var):
    var = jnp.maximum(var, 0.0)
    return jnp.where(var == 0.0, 1.0, lax.rsqrt(var))


def _message_core(V, Vj, Ee, W1s, W2, W3, vec, bn, knb):
    Sv = jnp.sum(V, axis=1, keepdims=True)
    Qv = jnp.sum(V * V, axis=1, keepdims=True)
    Sj = jnp.sum(Vj, axis=1, keepdims=True) + jnp.sum(Ee, axis=1, keepdims=True)
    Qj = jnp.sum(Vj * Vj, axis=1, keepdims=True) + jnp.sum(Ee * Ee, axis=1, keepdims=True)

    S = Sv.reshape(bn, 1, 1) + Sj.reshape(bn, knb, 1)
    Q = Qv.reshape(bn, 1, 1) + Qj.reshape(bn, knb, 1)
    mu = S * (1.0 / (3 * D))
    inv = _inv_std((Q - (3 * D) * mu * mu) * (1.0 / (3 * D - 1)))

    Pi = jnp.dot(V, W1s[0:D], preferred_element_type=jnp.float32)
    Pj = jnp.dot(Vj, W1s[D:2 * D], preferred_element_type=jnp.float32)
    Pe = jnp.dot(Ee, W1s[2 * D:3 * D], preferred_element_type=jnp.float32)
    P = Pi.reshape(bn, 1, D) + (Pj + Pe).reshape(bn, knb, D)

    u1 = vec[0:1, :].reshape(1, 1, D)
    c1 = vec[1:2, :].reshape(1, 1, D)
    h1 = P * inv - (mu * inv) * u1 + c1
    z1 = (h1 * jax.nn.sigmoid(h1)).reshape(bn * knb, D)
    h2 = jnp.dot(z1, W2, preferred_element_type=jnp.float32) + vec[2:3, :]
    z2 = h2 * jax.nn.sigmoid(h2)
    return jnp.dot(z2, W3, preferred_element_type=jnp.float32) + vec[3:4, :]


def _p1_body(bn, knb, v_ref, vj_ref, e_ref, w1_ref, w2_ref, w3_ref,
             wf1_ref, wf2_ref, bf1_ref, vec_ref, out_ref):
    V = v_ref[...]
    vec = vec_ref[...]
    msg = _message_core(V, vj_ref[...], e_ref[...], w1_ref[...],
                        w2_ref[...], w3_ref[...], vec, bn, knb)
    Mv = jnp.sum(msg.reshape(bn, knb, D), axis=1)
    Mv = vec[4:5, :] * Mv

    x = V + Mv
    mu2 = jnp.sum(x, axis=1, keepdims=True) * (1.0 / D)
    d = x - mu2
    inv2 = _inv_std(jnp.sum(d * d, axis=1, keepdims=True) * (1.0 / (D - 1)))
    Vn = vec[5:6, :] * (d * inv2) + vec[6:7, :]

    hf = jnp.dot(Vn, wf1_ref[...], preferred_element_type=jnp.float32) + bf1_ref[...]
    zf = hf * jax.nn.sigmoid(hf)
    f = jnp.dot(zf, wf2_ref[...], preferred_element_type=jnp.float32) + vec[8:9, :]
    out_ref[...] = Vn + vec[7:8, :] * f


def _p2_body(bn, knb, v_ref, vj_ref, e_ref, w1_ref, w2_ref, w3_ref,
             vec_ref, out_ref):
    vec = vec_ref[...]
    Ee = e_ref[...]
    msg = _message_core(v_ref[...], vj_ref[...], Ee, w1_ref[...],
                        w2_ref[...], w3_ref[...], vec, bn, knb)
    out_ref[...] = Ee + vec[4:5, :] * msg


def _phase1_call(V2d, Vj, Ef, W1s, W2, W3, Wf1, Wf2, bf1, vec, bn, interpret=False):
    n = V2d.shape[0]
    knb = Ef.shape[0] // n
    m = bn * knb
    full = lambda shape: pl.BlockSpec(shape, lambda i: (0, 0))
    return pl.pallas_call(
        functools.partial(_p1_body, bn, knb),
        grid=(n // bn,),
        in_specs=[
            pl.BlockSpec((bn, D), lambda i: (i, 0)),
            pl.BlockSpec((m, D), lambda i: (i, 0)),
            pl.BlockSpec((m, D), lambda i: (i, 0)),
            full((3 * D, D)), full((D, D)), full((D, D)),
            full((D, 4 * D)), full((4 * D, D)), full((1, 4 * D)),
            full((16, D)),
        ],
        out_specs=pl.BlockSpec((bn, D), lambda i: (i, 0)),
        out_shape=jax.ShapeDtypeStruct((n, D), jnp.float32),
        interpret=interpret,
    )(V2d, Vj, Ef, W1s, W2, W3, Wf1, Wf2, bf1, vec)


def _phase2_call(V2d, Vj, Ef, W1s, W2, W3, vec, bn, interpret=False):
    n = V2d.shape[0]
    knb = Ef.shape[0] // n
    m = bn * knb
    full = lambda shape: pl.BlockSpec(shape, lambda i: (0, 0))
    return pl.pallas_call(
        functools.partial(_p2_body, bn, knb),
        grid=(n // bn,),
        in_specs=[
            pl.BlockSpec((bn, D), lambda i: (i, 0)),
            pl.BlockSpec((m, D), lambda i: (i, 0)),
            pl.BlockSpec((m, D), lambda i: (i, 0)),
            full((3 * D, D)), full((D, D)), full((D, D)),
            full((16, D)),
        ],
        out_specs=pl.BlockSpec((m, D), lambda i: (i, 0)),
        out_shape=jax.ShapeDtypeStruct((n * knb, D), jnp.float32),
        interpret=interpret,
    )(V2d, Vj, Ef, W1s, W2, W3, vec)


def kernel(V, E, K, t, nbr_mask, params):
    Z, N, knb, _ = E.shape
    V2d = V.reshape(N, D)
    Ef = E.reshape(N * knb, D)
    B = N * knb
    Bp = -(-B // (32 * _CH)) * (32 * _CH)
    Kf = K.reshape(B).astype(jnp.int32)
    idx2d = jnp.concatenate([Kf, jnp.zeros(Bp - B, jnp.int32)]).reshape(-1, _CH)

    t0 = t.reshape(D)
    g1, be1, a1 = _adaln_host(params['node_msgr_norm'], t0)
    g2, b2, a2 = _adaln_host(params['ffn_norm'], t0)
    g3, be3, a3 = _adaln_host(params['edge_msgr_norm'], t0)

    W1s, u1, c1, W2, b2w, W3, b3w = _fold_messenger(g1, be1, params['node_msgr'])
    W1es, u1e, c1e, W2e, b2we, W3e, b3we = _fold_messenger(g3, be3, params['edge_msgr'])
    (Wf1, bf1), (Wf2, bf2) = params['ffn']

    zed = jnp.zeros((D,), jnp.float32)
    vec1 = jnp.stack([u1, c1, b2w, b3w, a1, g2, b2, a2, bf2] + [zed] * 7)
    vec2 = jnp.stack([u1e, c1e, b2we, b3we, a3] + [zed] * 11)

    bn = 400
    Vj = _sc_gather(V2d, idx2d)
    V2 = _phase1_call(V2d, Vj, Ef, W1s, W2, W3, Wf1, Wf2,
                      bf1.reshape(1, 4 * D), vec1, bn)
    Vj2 = _sc_gather(V2, idx2d)
    Eo = _phase2_call(V2, Vj2, Ef, W1es, W2e, W3e, vec2, bn)

    return (V2.reshape(Z, N, D), Eo.reshape(Z, N, knb, D))


# speedup vs baseline: 2.2374x; 2.2374x over previous
import functools

import jax
import jax.numpy as jnp
from jax import lax
from jax.experimental import pallas as pl
from jax.experimental.pallas import tpu as pltpu
from jax.experimental.pallas import tpu_sc as plsc

D = 128


def _mlp_host(layers, x):
    for W, b in layers[:-1]:
        x = jax.nn.silu(x @ W + b)
    W, b = layers[-1]
    return x @ W + b


def _adaln_host(p, x):
    gb = _mlp_host(p['gb'], x)
    gamma, beta = jnp.split(gb, 2, axis=-1)
    alpha = _mlp_host(p['alpha'], x)
    return gamma, beta, alpha


def _fold_messenger(gamma, beta, mlp):
    (W1, b1), (W2, b2), (W3, b3) = mlp
    W1s = gamma[:, None] * W1
    u1 = jnp.sum(W1s, axis=0)
    c1 = beta @ W1 + b1
    return W1s, u1, c1, W2, b2, W3, b3


_CH = 128


def _sc_gather(table, idx2d):
    R = table.shape[0]
    n_slabs = idx2d.shape[0]
    info = plsc.get_sparse_core_info()
    nc, ns = info.num_cores, info.num_subcores
    nw = nc * ns
    n_ch = n_slabs // nw
    B = n_slabs * _CH
    rows_per = ((R // ns) + 7) & ~7
    last = R - rows_per * (ns - 1)
    assert last > 0 and last % 8 == 0 and rows_per % 8 == 0
    mesh = plsc.VectorSubcoreMesh(core_axis_name="c", subcore_axis_name="s")

    @functools.partial(
        pl.kernel,
        out_type=jax.ShapeDtypeStruct((B, D), jnp.float32),
        mesh=mesh,
        scratch_types=[
            pltpu.VMEM_SHARED((R, D), jnp.float32),
            pltpu.VMEM((n_ch, _CH), jnp.int32),
            pltpu.VMEM((2, _CH, D), jnp.float32),
            pltpu.SemaphoreType.DMA,
            pltpu.SemaphoreType.DMA,
        ],
    )
    def gather_k(table_hbm, idx_hbm, out_hbm, tbl_s, idx_v, rows_v, s0, s1):
        sid = lax.axis_index("s")
        wid = sid * nc + lax.axis_index("c")
        sems = (s0, s1)

        @pl.when(sid < ns - 1)
        def _():
            pltpu.sync_copy(table_hbm.at[pl.ds(sid * rows_per, rows_per)],
                            tbl_s.at[pl.ds(sid * rows_per, rows_per)])

        @pl.when(sid == ns - 1)
        def _():
            pltpu.sync_copy(table_hbm.at[pl.ds((ns - 1) * rows_per, last)],
                            tbl_s.at[pl.ds((ns - 1) * rows_per, last)])

        plsc.subcore_barrier()

        pltpu.sync_copy(idx_hbm.at[pl.ds(wid * n_ch, n_ch)], idx_v)

        def out_slab(c):
            return out_hbm.at[pl.ds((wid * n_ch + c) * _CH, _CH)]

        for c in range(n_ch):
            b = c % 2
            if c >= 2:
                pltpu.make_async_copy(rows_v.at[b], out_slab(c - 2),
                                      sems[b]).wait()
            pltpu.sync_copy(tbl_s.at[idx_v.at[c]], rows_v.at[b])
            pltpu.async_copy(rows_v.at[b], out_slab(c), sems[b])
        for c in (n_ch - 2, n_ch - 1):
            pltpu.make_async_copy(rows_v.at[c % 2], out_slab(c),
                                  sems[c % 2]).wait()

    return gather_k(table, idx2d)


def _inv_std(var):
    var = jnp.maximum(var, 0.0)
    return jnp.where(var == 0.0, 1.0, lax.rsqrt(var))


def _message_core(V, Vj, Ee, W1s, W2, W3, vec, bn, knb):
    Sv = jnp.sum(V, axis=1, keepdims=True)
    Qv = jnp.sum(V * V, axis=1, keepdims=True)
    Sj = jnp.sum(Vj, axis=1, keepdims=True) + jnp.sum(Ee, axis=1, keepdims=True)
    Qj = jnp.sum(Vj * Vj, axis=1, keepdims=True) + jnp.sum(Ee * Ee, axis=1, keepdims=True)

    S = Sv.reshape(bn, 1, 1) + Sj.reshape(bn, knb, 1)
    Q = Qv.reshape(bn, 1, 1) + Qj.reshape(bn, knb, 1)
    mu = S * (1.0 / (3 * D))
    inv = _inv_std((Q - (3 * D) * mu * mu) * (1.0 / (3 * D - 1)))

    Pi = jnp.dot(V, W1s[0:D], preferred_element_type=jnp.float32)
    Pj = jnp.dot(Vj, W1s[D:2 * D], preferred_element_type=jnp.float32)
    Pe = jnp.dot(Ee, W1s[2 * D:3 * D], preferred_element_type=jnp.float32)
    P = Pi.reshape(bn, 1, D) + (Pj + Pe).reshape(bn, knb, D)

    u1 = vec[0:1, :].reshape(1, 1, D)
    c1 = vec[1:2, :].reshape(1, 1, D)
    h1 = P * inv - (mu * inv) * u1 + c1
    z1 = (h1 * jax.nn.sigmoid(h1)).reshape(bn * knb, D)
    h2 = jnp.dot(z1, W2, preferred_element_type=jnp.float32) + vec[2:3, :]
    z2 = h2 * jax.nn.sigmoid(h2)
    return jnp.dot(z2, W3, preferred_element_type=jnp.float32) + vec[3:4, :]


def _p1_body(bn, knb, v_ref, vj_ref, e_ref, w1_ref, w2_ref, w3_ref,
             wf1_ref, wf2_ref, bf1_ref, vec_ref, out_ref):
    V = v_ref[...]
    vec = vec_ref[...]
    msg = _message_core(V, vj_ref[...], e_ref[...], w1_ref[...],
                        w2_ref[...], w3_ref[...], vec, bn, knb)
    Mv = jnp.sum(msg.reshape(bn, knb, D), axis=1)
    Mv = vec[4:5, :] * Mv

    x = V + Mv
    mu2 = jnp.sum(x, axis=1, keepdims=True) * (1.0 / D)
    d = x - mu2
    inv2 = _inv_std(jnp.sum(d * d, axis=1, keepdims=True) * (1.0 / (D - 1)))
    Vn = vec[5:6, :] * (d * inv2) + vec[6:7, :]

    hf = jnp.dot(Vn, wf1_ref[...], preferred_element_type=jnp.float32) + bf1_ref[...]
    zf = hf * jax.nn.sigmoid(hf)
    f = jnp.dot(zf, wf2_ref[...], preferred_element_type=jnp.float32) + vec[8:9, :]
    out_ref[...] = Vn + vec[7:8, :] * f


def _p2_body(bn, knb, v_ref, vj_ref, e_ref, w1_ref, w2_ref, w3_ref,
             vec_ref, out_ref):
    vec = vec_ref[...]
    Ee = e_ref[...]
    msg = _message_core(v_ref[...], vj_ref[...], Ee, w1_ref[...],
                        w2_ref[...], w3_ref[...], vec, bn, knb)
    out_ref[...] = Ee + vec[4:5, :] * msg


def _phase1_call(V2d, Vj, Ef, W1s, W2, W3, Wf1, Wf2, bf1, vec, bn, interpret=False):
    n = V2d.shape[0]
    knb = Ef.shape[0] // n
    m = bn * knb
    full = lambda shape: pl.BlockSpec(shape, lambda i: (0, 0))
    return pl.pallas_call(
        functools.partial(_p1_body, bn, knb),
        grid=(n // bn,),
        in_specs=[
            pl.BlockSpec((bn, D), lambda i: (i, 0)),
            pl.BlockSpec((m, D), lambda i: (i, 0)),
            pl.BlockSpec((m, D), lambda i: (i, 0)),
            full((3 * D, D)), full((D, D)), full((D, D)),
            full((D, 4 * D)), full((4 * D, D)), full((1, 4 * D)),
            full((16, D)),
        ],
        out_specs=pl.BlockSpec((bn, D), lambda i: (i, 0)),
        out_shape=jax.ShapeDtypeStruct((n, D), jnp.float32),
        interpret=interpret,
    )(V2d, Vj, Ef, W1s, W2, W3, Wf1, Wf2, bf1, vec)


def _phase2_call(V2d, Vj, Ef, W1s, W2, W3, vec, bn, interpret=False):
    n = V2d.shape[0]
    knb = Ef.shape[0] // n
    m = bn * knb
    full = lambda shape: pl.BlockSpec(shape, lambda i: (0, 0))
    return pl.pallas_call(
        functools.partial(_p2_body, bn, knb),
        grid=(n // bn,),
        in_specs=[
            pl.BlockSpec((bn, D), lambda i: (i, 0)),
            pl.BlockSpec((m, D), lambda i: (i, 0)),
            pl.BlockSpec((m, D), lambda i: (i, 0)),
            full((3 * D, D)), full((D, D)), full((D, D)),
            full((16, D)),
        ],
        out_specs=pl.BlockSpec((m, D), lambda i: (i, 0)),
        out_shape=jax.ShapeDtypeStruct((n * knb, D), jnp.float32),
        interpret=interpret,
    )(V2d, Vj, Ef, W1s, W2, W3, vec)


def kernel(V, E, K, t, nbr_mask, params):
    Z, N, knb, _ = E.shape
    V2d = V.reshape(N, D)
    Ef = E.reshape(N * knb, D)
    B = N * knb
    Bp = -(-B // (32 * _CH)) * (32 * _CH)
    Kf = K.reshape(B).astype(jnp.int32)
    idx2d = jnp.concatenate([Kf, jnp.zeros(Bp - B, jnp.int32)]).reshape(-1, _CH)

    t0 = t.reshape(D)
    g1, be1, a1 = _adaln_host(params['node_msgr_norm'], t0)
    g2, b2, a2 = _adaln_host(params['ffn_norm'], t0)
    g3, be3, a3 = _adaln_host(params['edge_msgr_norm'], t0)

    W1s, u1, c1, W2, b2w, W3, b3w = _fold_messenger(g1, be1, params['node_msgr'])
    W1es, u1e, c1e, W2e, b2we, W3e, b3we = _fold_messenger(g3, be3, params['edge_msgr'])
    (Wf1, bf1), (Wf2, bf2) = params['ffn']

    zed = jnp.zeros((D,), jnp.float32)
    vec1 = jnp.stack([u1, c1, b2w, b3w, a1, g2, b2, a2, bf2] + [zed] * 7)
    vec2 = jnp.stack([u1e, c1e, b2we, b3we, a3] + [zed] * 11)

    bn = 400
    Vj = _sc_gather(V2d, idx2d)
    V2 = _phase1_call(V2d, Vj, Ef, W1s, W2, W3, Wf1, Wf2,
                      bf1.reshape(1, 4 * D), vec1, bn)
    Vj2 = _sc_gather(V2, idx2d)
    Eo = _phase2_call(V2, Vj2, Ef, W1es, W2e, W3e, vec2, bn)

    return (V2.reshape(Z, N, D), Eo.reshape(Z, N, knb, D))
